# baseline (speedup 1.0000x reference)
import functools

import jax
import jax.numpy as jnp
from jax import lax
from jax.experimental import pallas as pl
from jax.experimental.pallas import tpu as pltpu
from jax.experimental.pallas import tpu_sc as plsc

T, D, F, E, K = 2048, 1024, 2048, 8, 2
A = T * K
TB = 1024
LOG_TB = 10
NBLK = 12
NPAD = NBLK * TB
FF = 512
NF = F // FF
TR = 256

NC, NS = 2, 16
NW = NC * NS
ROWS_PER_TILE = NPAD // NW
GCH = 32
TOK_PER_TILE = T // NW
CCH = 32

_SC_PARAMS = pltpu.CompilerParams(needs_layout_passes=False)


@functools.cache
def _sc_mesh():
    return plsc.VectorSubcoreMesh(
        core_axis_name="c", subcore_axis_name="s",
        num_cores=NC, num_subcores=NS)


def _router_body(x_ref, wr_ref, i1_ref, i2_ref, p1_ref, p2_ref):
    x = x_ref[...]
    wr = wr_ref[...]
    scores = jax.lax.dot_general(
        x, wr, (((1,), (1,)), ((), ())), preferred_element_type=jnp.float32)
    iota = jax.lax.broadcasted_iota(jnp.int32, scores.shape, 1)
    m1 = jnp.max(scores, axis=1, keepdims=True)
    i1 = jnp.min(jnp.where(scores == m1, iota, E), axis=1, keepdims=True)
    masked = jnp.where(iota == i1, -jnp.inf, scores)
    m2 = jnp.max(masked, axis=1, keepdims=True)
    i2 = jnp.min(jnp.where(masked == m2, iota, E), axis=1, keepdims=True)
    e21 = jnp.exp(m2 - m1)
    p1_ref[...] = 1.0 / (1.0 + e21)
    p2_ref[...] = e21 / (1.0 + e21)
    i1_ref[...] = i1
    i2_ref[...] = i2


def _router(x, w_router):
    out_shapes = (
        jax.ShapeDtypeStruct((T, 1), jnp.int32),
        jax.ShapeDtypeStruct((T, 1), jnp.int32),
        jax.ShapeDtypeStruct((T, 1), jnp.float32),
        jax.ShapeDtypeStruct((T, 1), jnp.float32),
    )
    o_spec = pl.BlockSpec((TR, 1), lambda i: (i, 0))
    return pl.pallas_call(
        _router_body,
        grid=(T // TR,),
        in_specs=[
            pl.BlockSpec((TR, D), lambda i: (i, 0)),
            pl.BlockSpec((E, D), lambda i: (0, 0)),
        ],
        out_specs=(o_spec, o_spec, o_spec, o_spec),
        out_shape=out_shapes,
    )(x, w_router)


def _route_body(i1_hbm, i2_hbm, p1_hbm, p2_hbm, x_hbm,
                xs_hbm, pp_hbm, pos1_hbm, pos2_hbm, be_hbm, act_hbm, bc_hbm,
                i1_v, i2_v, p1_v, p2_v, dest_v, tok_v, prob_v, be_v, act_v,
                bc_v, cnt_tab, rowbuf, rowbuf2, sem, sem2):
    c = lax.axis_index("c")
    s = lax.axis_index("s")
    wid = s * NC + c
    iota16 = lax.iota(jnp.int32, 16)
    z16i = jnp.zeros((16,), jnp.int32)
    lane15 = jnp.full((16,), 15, jnp.int32)

    def _take16(vec, idx):
        return lax.gather(
            vec, idx[:, None],
            lax.GatherDimensionNumbers(
                offset_dims=(), collapsed_slice_dims=(0,),
                start_index_map=(0,)),
            (1,), mode=lax.GatherScatterMode.PROMISE_IN_BOUNDS)

    def _bcast_last(vec):
        return _take16(vec, lane15)

    pltpu.sync_copy(i1_hbm, i1_v)
    pltpu.sync_copy(i2_hbm, i2_v)

    for e in range(E):
        cnt_tab[pl.ds(e * 16, 16)] = z16i

    def _hstep(i, _):
        qv = iota16 * (T // 16) + i
        for src in (i1_v, i2_v):
            v = plsc.load_gather(src, [qv])
            cidx = v * 16 + iota16
            cc = plsc.load_gather(cnt_tab, [cidx])
            plsc.store_scatter(cnt_tab, [cidx], cc + 1)
        return 0

    lax.fori_loop(0, T // 16, _hstep, 0)

    counts = z16i
    for e in range(E):
        ct = cnt_tab[pl.ds(e * 16, 16)]
        pcs = plsc.cumsum(ct)
        counts = jnp.where(iota16 == e, _bcast_last(pcs), counts)

    nblk = lax.shift_right_logical(counts + (TB - 1), LOG_TB)
    cumblk = plsc.cumsum(nblk)
    pad_start = lax.shift_left(cumblk - nblk, LOG_TB)
    totblk_vec = _take16(cumblk, jnp.full((16,), E - 1, jnp.int32))
    used_rows = lax.shift_left(
        jnp.sum(jnp.where(iota16 == 0, totblk_vec, z16i)), LOG_TB)

    be = z16i
    for e in range(E):
        ce = _take16(cumblk, jnp.full((16,), e, jnp.int32))
        be = be + (iota16 >= ce).astype(jnp.int32)
    last_used = jnp.maximum(totblk_vec - 1, z16i)
    be_clamped = jnp.where(iota16 < totblk_vec, be, _take16(be, last_used))
    active = jnp.where(iota16 < totblk_vec, iota16,
                       jnp.full((16,), -1, jnp.int32))
    be_v[pl.ds(0, 16)] = be_clamped
    act_v[pl.ds(0, 16)] = active
    bc_v[pl.ds(0, 16)] = jnp.where(iota16 < totblk_vec, iota16, last_used)

    for e in range(E):
        ct = cnt_tab[pl.ds(e * 16, 16)]
        excl = plsc.cumsum(ct) - ct
        pse = _take16(pad_start, jnp.full((16,), e, jnp.int32))
        cnt_tab[pl.ds(e * 16, 16)] = pse + excl

    def _zstep(j, _):
        tok_v[pl.ds(j * 16, 16)] = z16i
        prob_v[pl.ds(j * 16, 16)] = jnp.zeros((16,), jnp.float32)
        return 0
    lax.fori_loop(0, NPAD // 16, _zstep, 0)

    pltpu.sync_copy(p1_hbm, p1_v)
    pltpu.sync_copy(p2_hbm, p2_v)

    def _dstep(i, _):
        qv = iota16 * (T // 16) + i
        for qoff, src, psrc in ((0, i1_v, p1_v), (T, i2_v, p2_v)):
            v = plsc.load_gather(src, [qv])
            cidx = v * 16 + iota16
            b = plsc.load_gather(cnt_tab, [cidx])
            plsc.store_scatter(cnt_tab, [cidx], b + 1)
            plsc.store_scatter(dest_v, [qv + qoff], b)
            plsc.store_scatter(tok_v, [b], qv)
            p = plsc.load_gather(psrc, [qv])
            plsc.store_scatter(prob_v, [b], p)
        return 0

    lax.fori_loop(0, T // 16, _dstep, 0)

    @pl.when(wid == 0)
    def _emit_tables():
        pltpu.sync_copy(prob_v, pp_hbm)
        pltpu.sync_copy(dest_v.at[pl.ds(0, T)], pos1_hbm)
        pltpu.sync_copy(dest_v.at[pl.ds(T, T)], pos2_hbm)
        pltpu.sync_copy(be_v, be_hbm)
        pltpu.sync_copy(act_v, act_hbm)
        pltpu.sync_copy(bc_v, bc_hbm)

    base_row = wid * ROWS_PER_TILE
    nch = ROWS_PER_TILE // GCH
    bufs = (rowbuf, rowbuf2)
    sems = (sem, sem2)

    def _fire(ch):
        st = base_row + ch * GCH

        @pl.when(st < used_rows)
        def _f():
            pltpu.async_copy(
                x_hbm.at[tok_v.at[pl.ds(st, GCH)]], bufs[ch % 2], sems[ch % 2])

    _fire(0)
    for ch in range(nch):
        st = base_row + ch * GCH

        @pl.when(st < used_rows)
        def _w(ch=ch, st=st):
            pltpu.make_async_copy(
                x_hbm.at[tok_v.at[pl.ds(st, GCH)]], bufs[ch % 2],
                sems[ch % 2]).wait()
        if ch + 1 < nch:
            _fire(ch + 1)

        @pl.when(st < used_rows)
        def _o(ch=ch, st=st):
            pltpu.sync_copy(bufs[ch % 2], xs_hbm.at[pl.ds(st, GCH)])


def _route_sc(i1, i2, p1, p2, x):
    f = pl.kernel(
        _route_body,
        out_type=(
            jax.ShapeDtypeStruct((NPAD, D), jnp.float32),
            jax.ShapeDtypeStruct((NPAD,), jnp.float32),
            jax.ShapeDtypeStruct((T,), jnp.int32),
            jax.ShapeDtypeStruct((T,), jnp.int32),
            jax.ShapeDtypeStruct((16,), jnp.int32),
            jax.ShapeDtypeStruct((16,), jnp.int32),
            jax.ShapeDtypeStruct((16,), jnp.int32),
        ),
        mesh=_sc_mesh(),
        compiler_params=_SC_PARAMS,
        scratch_types=[
            pltpu.VMEM((T,), jnp.int32),
            pltpu.VMEM((T,), jnp.int32),
            pltpu.VMEM((T,), jnp.float32),
            pltpu.VMEM((T,), jnp.float32),
            pltpu.VMEM((A,), jnp.int32),
            pltpu.VMEM((NPAD,), jnp.int32),
            pltpu.VMEM((NPAD,), jnp.float32),
            pltpu.VMEM((16,), jnp.int32),
            pltpu.VMEM((16,), jnp.int32),
            pltpu.VMEM((16,), jnp.int32),
            pltpu.VMEM((E * 16,), jnp.int32),
            pltpu.VMEM((GCH, D), jnp.float32),
            pltpu.VMEM((GCH, D), jnp.float32),
            pltpu.SemaphoreType.DMA,
            pltpu.SemaphoreType.DMA,
        ],
    )
    return f(i1, i2, p1, p2, x)


def _expert_body(be_ref, act_ref, bc_ref, xs_ref, pr_ref, wg_ref, wu_ref,
                 wd_ref, out_ref):
    b = pl.program_id(0)
    f = pl.program_id(1)

    @pl.when(f == 0)
    def _init():
        out_ref[...] = jnp.zeros_like(out_ref)

    @pl.when(act_ref[b] >= 0)
    def _compute():
        x = xs_ref[...].astype(jnp.bfloat16)
        wg = wg_ref[0].astype(jnp.bfloat16)
        wu = wu_ref[0].astype(jnp.bfloat16)
        g = jax.lax.dot_general(
            x, wg, (((1,), (1,)), ((), ())), preferred_element_type=jnp.float32)
        u = jax.lax.dot_general(
            x, wu, (((1,), (1,)), ((), ())), preferred_element_type=jnp.float32)
        h = (g * jax.nn.sigmoid(g) * u).astype(jnp.bfloat16)
        wd = wd_ref[0].astype(jnp.bfloat16)
        acc = jax.lax.dot_general(
            h, wd, (((1,), (1,)), ((), ())), preferred_element_type=jnp.float32)
        out_ref[...] += acc

    @pl.when(f == NF - 1)
    def _scale():
        out_ref[...] *= pr_ref[0]


def _expert_mm(block_expert, active, bclamp, xs, probs_pad,
               w_gate, w_up, w_down):
    def fz(b, f, ac):
        return jnp.where(ac[b] >= 0, f, NF - 1)

    grid_spec = pltpu.PrefetchScalarGridSpec(
        num_scalar_prefetch=3,
        grid=(NBLK, NF),
        in_specs=[
            pl.BlockSpec((TB, D), lambda b, f, be, ac, bc: (bc[b], 0)),
            pl.BlockSpec((1, TB, 1), lambda b, f, be, ac, bc: (bc[b], 0, 0)),
            pl.BlockSpec(
                (1, FF, D), lambda b, f, be, ac, bc: (be[b], fz(b, f, ac), 0)),
            pl.BlockSpec(
                (1, FF, D), lambda b, f, be, ac, bc: (be[b], fz(b, f, ac), 0)),
            pl.BlockSpec(
                (1, D, FF), lambda b, f, be, ac, bc: (be[b], 0, fz(b, f, ac))),
        ],
        out_specs=pl.BlockSpec((TB, D), lambda b, f, be, ac, bc: (b, 0)),
    )
    return pl.pallas_call(
        _expert_body,
        grid_spec=grid_spec,
        out_shape=jax.ShapeDtypeStruct((NPAD, D), jnp.float32),
        compiler_params=pltpu.CompilerParams(
            dimension_semantics=("arbitrary", "arbitrary")),
    )(block_expert, active, bclamp, xs, probs_pad.reshape(NBLK, TB, 1),
      w_gate, w_up, w_down)


def _combine_body(outs_hbm, pos1_hbm, pos2_hbm, y_hbm,
                  idx1_v, idx2_v, buf1, buf2, sem1, sem2):
    c = lax.axis_index("c")
    s = lax.axis_index("s")
    wid = s * NC + c
    base = wid * TOK_PER_TILE
    pltpu.sync_copy(pos1_hbm.at[pl.ds(base, TOK_PER_TILE)], idx1_v)
    pltpu.sync_copy(pos2_hbm.at[pl.ds(base, TOK_PER_TILE)], idx2_v)
    for ch in range(TOK_PER_TILE // CCH):
        off = ch * CCH
        cp1 = pltpu.async_copy(
            outs_hbm.at[idx1_v.at[pl.ds(off, CCH)]], buf1, sem1)
        cp2 = pltpu.async_copy(
            outs_hbm.at[idx2_v.at[pl.ds(off, CCH)]], buf2, sem2)
        cp1.wait()
        cp2.wait()

        def add_row(i, _):
            def add_sl(j, _):
                for u in range(4):
                    o = pl.ds(j * 64 + u * 16, 16)
                    buf1[i, o] = buf1[i, o] + buf2[i, o]
                return 0
            return lax.fori_loop(0, D // 64, add_sl, 0)

        lax.fori_loop(0, CCH, add_row, 0)
        pltpu.sync_copy(buf1, y_hbm.at[pl.ds(base + off, CCH)])


def _combine_sc(outs, pos1, pos2):
    f = pl.kernel(
        _combine_body,
        out_type=jax.ShapeDtypeStruct((T, D), jnp.float32),
        mesh=_sc_mesh(),
        compiler_params=_SC_PARAMS,
        scratch_types=[
            pltpu.VMEM((TOK_PER_TILE,), jnp.int32),
            pltpu.VMEM((TOK_PER_TILE,), jnp.int32),
            pltpu.VMEM((CCH, D), jnp.float32),
            pltpu.VMEM((CCH, D), jnp.float32),
            pltpu.SemaphoreType.DMA,
            pltpu.SemaphoreType.DMA,
        ],
    )
    return f(outs, pos1, pos2)


def kernel(x, W_router, W_gate, W_up, W_down):
    i1, i2, p1, p2 = _router(x, W_router)
    xs, probs_pad, pos1, pos2, block_expert, active, bclamp = _route_sc(
        i1.reshape(T), i2.reshape(T), p1.reshape(T), p2.reshape(T), x)
    outs = _expert_mm(block_expert, active, bclamp, xs, probs_pad,
                      W_gate, W_up, W_down)
    return _combine_sc(outs, pos1, pos2)


# speedup vs baseline: 1.4646x; 1.4646x over previous
import functools

import jax
import jax.numpy as jnp
from jax import lax
from jax.experimental import pallas as pl
from jax.experimental.pallas import tpu as pltpu
from jax.experimental.pallas import tpu_sc as plsc

T, D, F, E, K = 2048, 1024, 2048, 8, 2
A = T * K
TB = 512
LOG_TB = 9
NBLK = 16
NPAD = NBLK * TB
FF = 512
NF = F // FF
TR = 256

NC, NS = 2, 16
NW = NC * NS
ROWS_PER_TILE = NPAD // NW
GCH = 32
TOK_PER_TILE = T // NW
CCH = 32

_SC_PARAMS = pltpu.CompilerParams(needs_layout_passes=False)


@functools.cache
def _sc_mesh():
    return plsc.VectorSubcoreMesh(
        core_axis_name="c", subcore_axis_name="s",
        num_cores=NC, num_subcores=NS)


def _router_body(x_ref, wr_ref, i1_ref, i2_ref, p1_ref, p2_ref):
    x = x_ref[...]
    wr = wr_ref[...]
    scores = jax.lax.dot_general(
        x, wr, (((1,), (1,)), ((), ())), preferred_element_type=jnp.float32)
    iota = jax.lax.broadcasted_iota(jnp.int32, scores.shape, 1)
    m1 = jnp.max(scores, axis=1, keepdims=True)
    i1 = jnp.min(jnp.where(scores == m1, iota, E), axis=1, keepdims=True)
    masked = jnp.where(iota == i1, -jnp.inf, scores)
    m2 = jnp.max(masked, axis=1, keepdims=True)
    i2 = jnp.min(jnp.where(masked == m2, iota, E), axis=1, keepdims=True)
    e21 = jnp.exp(m2 - m1)
    p1_ref[...] = 1.0 / (1.0 + e21)
    p2_ref[...] = e21 / (1.0 + e21)
    i1_ref[...] = i1
    i2_ref[...] = i2


def _router(x, w_router):
    out_shapes = (
        jax.ShapeDtypeStruct((T, 1), jnp.int32),
        jax.ShapeDtypeStruct((T, 1), jnp.int32),
        jax.ShapeDtypeStruct((T, 1), jnp.float32),
        jax.ShapeDtypeStruct((T, 1), jnp.float32),
    )
    o_spec = pl.BlockSpec((TR, 1), lambda i: (i, 0))
    return pl.pallas_call(
        _router_body,
        grid=(T // TR,),
        in_specs=[
            pl.BlockSpec((TR, D), lambda i: (i, 0)),
            pl.BlockSpec((E, D), lambda i: (0, 0)),
        ],
        out_specs=(o_spec, o_spec, o_spec, o_spec),
        out_shape=out_shapes,
    )(x, w_router)


def _route_body(i1_hbm, i2_hbm, p1_hbm, p2_hbm, x_hbm,
                xs_hbm, pp_hbm, pos1_hbm, pos2_hbm, be_hbm, act_hbm, bc_hbm,
                i1_v, i2_v, p1_v, p2_v, dest_v, tok_v, prob_v, be_v, act_v,
                bc_v, cnt_tab, rowbuf, rowbuf2, sem, sem2):
    c = lax.axis_index("c")
    s = lax.axis_index("s")
    wid = s * NC + c
    iota16 = lax.iota(jnp.int32, 16)
    z16i = jnp.zeros((16,), jnp.int32)
    lane15 = jnp.full((16,), 15, jnp.int32)

    def _take16(vec, idx):
        return lax.gather(
            vec, idx[:, None],
            lax.GatherDimensionNumbers(
                offset_dims=(), collapsed_slice_dims=(0,),
                start_index_map=(0,)),
            (1,), mode=lax.GatherScatterMode.PROMISE_IN_BOUNDS)

    def _bcast_last(vec):
        return _take16(vec, lane15)

    pltpu.sync_copy(i1_hbm, i1_v)
    pltpu.sync_copy(i2_hbm, i2_v)

    for e in range(E):
        cnt_tab[pl.ds(e * 16, 16)] = z16i

    def _hstep(i, _):
        qv = iota16 * (T // 16) + i
        for src in (i1_v, i2_v):
            v = plsc.load_gather(src, [qv])
            cidx = v * 16 + iota16
            cc = plsc.load_gather(cnt_tab, [cidx])
            plsc.store_scatter(cnt_tab, [cidx], cc + 1)
        return 0

    lax.fori_loop(0, T // 16, _hstep, 0)

    counts = z16i
    for e in range(E):
        ct = cnt_tab[pl.ds(e * 16, 16)]
        pcs = plsc.cumsum(ct)
        counts = jnp.where(iota16 == e, _bcast_last(pcs), counts)

    nblk = lax.shift_right_logical(counts + (TB - 1), LOG_TB)
    cumblk = plsc.cumsum(nblk)
    pad_start = lax.shift_left(cumblk - nblk, LOG_TB)
    totblk_vec = _take16(cumblk, jnp.full((16,), E - 1, jnp.int32))
    used_rows = lax.shift_left(
        jnp.sum(jnp.where(iota16 == 0, totblk_vec, z16i)), LOG_TB)

    be = z16i
    for e in range(E):
        ce = _take16(cumblk, jnp.full((16,), e, jnp.int32))
        be = be + (iota16 >= ce).astype(jnp.int32)
    last_used = jnp.maximum(totblk_vec - 1, z16i)
    be_clamped = jnp.where(iota16 < totblk_vec, be, _take16(be, last_used))
    active = jnp.where(iota16 < totblk_vec, iota16,
                       jnp.full((16,), -1, jnp.int32))
    be_v[pl.ds(0, 16)] = be_clamped
    act_v[pl.ds(0, 16)] = active
    bc_v[pl.ds(0, 16)] = jnp.where(iota16 < totblk_vec, iota16, last_used)

    for e in range(E):
        ct = cnt_tab[pl.ds(e * 16, 16)]
        excl = plsc.cumsum(ct) - ct
        pse = _take16(pad_start, jnp.full((16,), e, jnp.int32))
        cnt_tab[pl.ds(e * 16, 16)] = pse + excl

    def _zstep(j, _):
        tok_v[pl.ds(j * 16, 16)] = z16i
        prob_v[pl.ds(j * 16, 16)] = jnp.zeros((16,), jnp.float32)
        return 0
    lax.fori_loop(0, NPAD // 16, _zstep, 0)

    pltpu.sync_copy(p1_hbm, p1_v)
    pltpu.sync_copy(p2_hbm, p2_v)

    def _dstep(i, _):
        qv = iota16 * (T // 16) + i
        for qoff, src, psrc in ((0, i1_v, p1_v), (T, i2_v, p2_v)):
            v = plsc.load_gather(src, [qv])
            cidx = v * 16 + iota16
            b = plsc.load_gather(cnt_tab, [cidx])
            plsc.store_scatter(cnt_tab, [cidx], b + 1)
            plsc.store_scatter(dest_v, [qv + qoff], b)
            plsc.store_scatter(tok_v, [b], qv)
            p = plsc.load_gather(psrc, [qv])
            plsc.store_scatter(prob_v, [b], p)
        return 0

    lax.fori_loop(0, T // 16, _dstep, 0)

    @pl.when(wid == 0)
    def _emit_tables():
        pltpu.sync_copy(prob_v, pp_hbm)
        pltpu.sync_copy(dest_v.at[pl.ds(0, T)], pos1_hbm)
        pltpu.sync_copy(dest_v.at[pl.ds(T, T)], pos2_hbm)
        pltpu.sync_copy(be_v, be_hbm)
        pltpu.sync_copy(act_v, act_hbm)
        pltpu.sync_copy(bc_v, bc_hbm)

    base_row = wid * ROWS_PER_TILE
    nch = ROWS_PER_TILE // GCH
    bufs = (rowbuf, rowbuf2)
    sems = (sem, sem2)

    def _fire(ch):
        st = base_row + ch * GCH

        @pl.when(st < used_rows)
        def _f():
            pltpu.async_copy(
                x_hbm.at[tok_v.at[pl.ds(st, GCH)]], bufs[ch % 2], sems[ch % 2])

    _fire(0)
    for ch in range(nch):
        st = base_row + ch * GCH

        @pl.when(st < used_rows)
        def _w(ch=ch, st=st):
            pltpu.make_async_copy(
                x_hbm.at[tok_v.at[pl.ds(st, GCH)]], bufs[ch % 2],
                sems[ch % 2]).wait()
        if ch + 1 < nch:
            _fire(ch + 1)

        @pl.when(st < used_rows)
        def _o(ch=ch, st=st):
            pltpu.sync_copy(bufs[ch % 2], xs_hbm.at[pl.ds(st, GCH)])


def _route_sc(i1, i2, p1, p2, x):
    f = pl.kernel(
        _route_body,
        out_type=(
            jax.ShapeDtypeStruct((NPAD, D), jnp.float32),
            jax.ShapeDtypeStruct((NPAD,), jnp.float32),
            jax.ShapeDtypeStruct((T,), jnp.int32),
            jax.ShapeDtypeStruct((T,), jnp.int32),
            jax.ShapeDtypeStruct((NBLK,), jnp.int32),
            jax.ShapeDtypeStruct((NBLK,), jnp.int32),
            jax.ShapeDtypeStruct((NBLK,), jnp.int32),
        ),
        mesh=_sc_mesh(),
        compiler_params=_SC_PARAMS,
        scratch_types=[
            pltpu.VMEM((T,), jnp.int32),
            pltpu.VMEM((T,), jnp.int32),
            pltpu.VMEM((T,), jnp.float32),
            pltpu.VMEM((T,), jnp.float32),
            pltpu.VMEM((A,), jnp.int32),
            pltpu.VMEM((NPAD,), jnp.int32),
            pltpu.VMEM((NPAD,), jnp.float32),
            pltpu.VMEM((16,), jnp.int32),
            pltpu.VMEM((16,), jnp.int32),
            pltpu.VMEM((16,), jnp.int32),
            pltpu.VMEM((E * 16,), jnp.int32),
            pltpu.VMEM((GCH, D), jnp.float32),
            pltpu.VMEM((GCH, D), jnp.float32),
            pltpu.SemaphoreType.DMA,
            pltpu.SemaphoreType.DMA,
        ],
    )
    return f(i1, i2, p1, p2, x)


def _expert_body(be_ref, act_ref, bc_ref, xs_ref, pr_ref, wg_ref, wu_ref,
                 wd_ref, out_ref):
    b = pl.program_id(0)
    f = pl.program_id(1)

    @pl.when(f == 0)
    def _init():
        out_ref[...] = jnp.zeros_like(out_ref)

    @pl.when(act_ref[b] >= 0)
    def _compute():
        x = xs_ref[...].astype(jnp.bfloat16)
        wg = wg_ref[0].astype(jnp.bfloat16)
        wu = wu_ref[0].astype(jnp.bfloat16)
        g = jax.lax.dot_general(
            x, wg, (((1,), (1,)), ((), ())), preferred_element_type=jnp.float32)
        u = jax.lax.dot_general(
            x, wu, (((1,), (1,)), ((), ())), preferred_element_type=jnp.float32)
        h = (g * jax.nn.sigmoid(g) * u).astype(jnp.bfloat16)
        wd = wd_ref[0].astype(jnp.bfloat16)
        acc = jax.lax.dot_general(
            h, wd, (((1,), (1,)), ((), ())), preferred_element_type=jnp.float32)
        out_ref[...] += acc

    @pl.when(f == NF - 1)
    def _scale():
        out_ref[...] *= pr_ref[0]


def _expert_mm(block_expert, active, bclamp, xs, probs_pad,
               w_gate, w_up, w_down):
    def fz(b, f, ac):
        return jnp.where(ac[b] >= 0, f, NF - 1)

    grid_spec = pltpu.PrefetchScalarGridSpec(
        num_scalar_prefetch=3,
        grid=(NBLK, NF),
        in_specs=[
            pl.BlockSpec((TB, D), lambda b, f, be, ac, bc: (bc[b], 0)),
            pl.BlockSpec((1, TB, 1), lambda b, f, be, ac, bc: (bc[b], 0, 0)),
            pl.BlockSpec(
                (1, FF, D), lambda b, f, be, ac, bc: (be[b], fz(b, f, ac), 0)),
            pl.BlockSpec(
                (1, FF, D), lambda b, f, be, ac, bc: (be[b], fz(b, f, ac), 0)),
            pl.BlockSpec(
                (1, D, FF), lambda b, f, be, ac, bc: (be[b], 0, fz(b, f, ac))),
        ],
        out_specs=pl.BlockSpec((TB, D), lambda b, f, be, ac, bc: (b, 0)),
    )
    return pl.pallas_call(
        _expert_body,
        grid_spec=grid_spec,
        out_shape=jax.ShapeDtypeStruct((NPAD, D), jnp.float32),
        compiler_params=pltpu.CompilerParams(
            dimension_semantics=("arbitrary", "arbitrary")),
    )(block_expert, active, bclamp, xs, probs_pad.reshape(NBLK, TB, 1),
      w_gate, w_up, w_down)


def _combine_body(outs_hbm, pos1_hbm, pos2_hbm, y_hbm,
                  idx1_v, idx2_v, buf1, buf2, sem1, sem2):
    c = lax.axis_index("c")
    s = lax.axis_index("s")
    wid = s * NC + c
    base = wid * TOK_PER_TILE
    pltpu.sync_copy(pos1_hbm.at[pl.ds(base, TOK_PER_TILE)], idx1_v)
    pltpu.sync_copy(pos2_hbm.at[pl.ds(base, TOK_PER_TILE)], idx2_v)
    for ch in range(TOK_PER_TILE // CCH):
        off = ch * CCH
        cp1 = pltpu.async_copy(
            outs_hbm.at[idx1_v.at[pl.ds(off, CCH)]], buf1, sem1)
        cp2 = pltpu.async_copy(
            outs_hbm.at[idx2_v.at[pl.ds(off, CCH)]], buf2, sem2)
        cp1.wait()
        cp2.wait()

        def add_row(i, _):
            def add_sl(j, _):
                for u in range(4):
                    o = pl.ds(j * 64 + u * 16, 16)
                    buf1[i, o] = buf1[i, o] + buf2[i, o]
                return 0
            return lax.fori_loop(0, D // 64, add_sl, 0)

        lax.fori_loop(0, CCH, add_row, 0)
        pltpu.sync_copy(buf1, y_hbm.at[pl.ds(base + off, CCH)])


def _combine_sc(outs, pos1, pos2):
    f = pl.kernel(
        _combine_body,
        out_type=jax.ShapeDtypeStruct((T, D), jnp.float32),
        mesh=_sc_mesh(),
        compiler_params=_SC_PARAMS,
        scratch_types=[
            pltpu.VMEM((TOK_PER_TILE,), jnp.int32),
            pltpu.VMEM((TOK_PER_TILE,), jnp.int32),
            pltpu.VMEM((CCH, D), jnp.float32),
            pltpu.VMEM((CCH, D), jnp.float32),
            pltpu.SemaphoreType.DMA,
            pltpu.SemaphoreType.DMA,
        ],
    )
    return f(outs, pos1, pos2)


def kernel(x, W_router, W_gate, W_up, W_down):
    i1, i2, p1, p2 = _router(x, W_router)
    xs, probs_pad, pos1, pos2, block_expert, active, bclamp = _route_sc(
        i1.reshape(T), i2.reshape(T), p1.reshape(T), p2.reshape(T), x)
    outs = _expert_mm(block_expert, active, bclamp, xs, probs_pad,
                      W_gate, W_up, W_down)
    return _combine_sc(outs, pos1, pos2)
